# Initial kernel scaffold; baseline (speedup 1.0000x reference)
import jax
import jax.numpy as jnp
from jax.experimental import pallas as pl


def kernel(inputs):
    raise NotImplementedError("write your pallas kernel here")


# speedup vs baseline: 11.4117x; 11.4117x over previous
import functools

import jax
import jax.numpy as jnp
from jax.experimental import pallas as pl

_TARGET_RATE = 0.5
_BLOCK_COLS = 256


def _select_mask_kernel(k, x_ref, o_ref):
    x = x_ref[...]
    u = jax.lax.bitcast_convert_type(jnp.abs(x), jnp.int32)
    rows = x.shape[0]
    prefix = jnp.zeros((1, x.shape[1]), dtype=jnp.int32)
    for b in range(30, -1, -1):
        cand = prefix | ((1 << b) - 1)
        cnt = jnp.sum((u <= cand).astype(jnp.int32), axis=0, keepdims=True)
        prefix = jnp.where(cnt >= k, prefix, prefix | (1 << b))
    o_ref[...] = jnp.where(u <= prefix, jnp.zeros_like(x), x)


def kernel(inputs):
    shape = inputs.shape
    d = shape[-1]
    rows = 1
    for s in shape[:-1]:
        rows *= s
    k = int(_TARGET_RATE * float(rows))
    x2 = inputs.reshape(rows, d)
    bc = min(_BLOCK_COLS, d)
    out = pl.pallas_call(
        functools.partial(_select_mask_kernel, k),
        grid=(d // bc,),
        in_specs=[pl.BlockSpec((rows, bc), lambda j: (0, j))],
        out_specs=pl.BlockSpec((rows, bc), lambda j: (0, j)),
        out_shape=jax.ShapeDtypeStruct((rows, d), inputs.dtype),
    )(x2)
    return out.reshape(shape)
